# Initial kernel scaffold; baseline (speedup 1.0000x reference)
import jax
import jax.numpy as jnp
from jax.experimental import pallas as pl


def kernel(xyz):
    raise NotImplementedError("write your pallas kernel here")


# speedup vs baseline: 1.0001x; 1.0001x over previous
import jax
import jax.numpy as jnp
from jax.experimental import pallas as pl

N_P = 512
K = 64


def _fps(xyz, n_points):
    B, N, _ = xyz.shape
    start = jnp.zeros((B,), dtype=jnp.int32)
    first = xyz[jnp.arange(B), start]
    dist = jnp.sum((xyz - first[:, None, :]) ** 2, axis=-1)

    def step(dist, _):
        idx = jnp.argmax(dist, axis=-1)
        pt = xyz[jnp.arange(B), idx]
        nd = jnp.sum((xyz - pt[:, None, :]) ** 2, axis=-1)
        return jnp.minimum(dist, nd), idx

    _, rest = jax.lax.scan(step, dist, None, length=n_points - 1)
    all_idx = jnp.concatenate([start[None, :], rest], axis=0).transpose(1, 0)
    centers = jax.vmap(lambda x, i: x[i])(xyz, all_idx)
    return centers


def _knn_idx(centers, xyz, k):
    c2 = jnp.sum(centers ** 2, axis=-1)[:, :, None]
    x2 = jnp.sum(xyz ** 2, axis=-1)[:, None, :]
    cross = jnp.einsum('bpc,bnc->bpn', centers, xyz)
    d2 = c2 + x2 - 2.0 * cross
    _, idx = jax.lax.top_k(-d2, k)
    return idx


def kernel(xyz):
    B = xyz.shape[0]
    centers = _fps(xyz, N_P)
    idx = _knn_idx(centers, xyz, K)
    flat = idx.reshape(B, N_P * K)
    gathered = jax.vmap(lambda x, i: x[i])(xyz, flat)
    patches = gathered.reshape(B, N_P, K, 3) - centers[:, :, None, :]
    return patches[None], centers[None]


# speedup vs baseline: 3.8609x; 3.8603x over previous
import functools

import jax
import jax.numpy as jnp
from jax import lax
from jax.experimental import pallas as pl
from jax.experimental.pallas import tpu as pltpu
from jax.experimental.pallas import tpu_sc as plsc

B = 8
N = 16384
P = 512
K = 64
PB = 128

NC = 2
NS = 16
NW = NC * NS
RPW = (B * P) // NW

_I32_BIG = 2 ** 30


def _fps_body(x0_ref, x1_ref, x2_ref, c0_ref, c1_ref, c2_ref, dist_ref):
    x0 = x0_ref[...]
    x1 = x1_ref[...]
    x2 = x2_ref[...]
    iota_n = lax.broadcasted_iota(jnp.int32, (B, N), 1)
    iota_p = lax.broadcasted_iota(jnp.int32, (B, P), 1)

    p0 = x0[:, 0:1]
    p1 = x1[:, 0:1]
    p2 = x2[:, 0:1]
    c0_ref[...] = jnp.where(iota_p == 0, p0, 0.0)
    c1_ref[...] = jnp.where(iota_p == 0, p1, 0.0)
    c2_ref[...] = jnp.where(iota_p == 0, p2, 0.0)
    d0 = x0 - p0
    d1 = x1 - p1
    d2 = x2 - p2
    dist_ref[...] = (d0 * d0 + d1 * d1) + d2 * d2

    def step(i, _):
        dist = dist_ref[...]
        m = jnp.max(dist, axis=1, keepdims=True)
        idx = jnp.min(jnp.where(dist == m, iota_n, N), axis=1, keepdims=True)
        sel = iota_n == idx
        p0 = jnp.sum(jnp.where(sel, x0, 0.0), axis=1, keepdims=True)
        p1 = jnp.sum(jnp.where(sel, x1, 0.0), axis=1, keepdims=True)
        p2 = jnp.sum(jnp.where(sel, x2, 0.0), axis=1, keepdims=True)
        hit = iota_p == i
        c0_ref[...] = jnp.where(hit, p0, c0_ref[...])
        c1_ref[...] = jnp.where(hit, p1, c1_ref[...])
        c2_ref[...] = jnp.where(hit, p2, c2_ref[...])
        e0 = x0 - p0
        e1 = x1 - p1
        e2 = x2 - p2
        nd = (e0 * e0 + e1 * e1) + e2 * e2
        dist_ref[...] = jnp.minimum(dist, nd)
        return 0

    lax.fori_loop(1, P, step, 0)


def _fps(x0, x1, x2):
    return pl.pallas_call(
        _fps_body,
        out_shape=[jax.ShapeDtypeStruct((B, P), jnp.float32)] * 3,
        scratch_shapes=[pltpu.VMEM((B, N), jnp.float32)],
    )(x0, x1, x2)


def _d2_body(c_ref, x_ref, c2_ref, x2_ref, d2_ref, thr_ref):
    cross = lax.dot_general(c_ref[0], x_ref[0], (((1,), (1,)), ((), ())))
    d2 = (c2_ref[0] + x2_ref[0]) - 2.0 * cross
    d2_ref[0] = d2
    m = d2[:, 0:128]
    for j in range(1, N // 128):
        m = jnp.minimum(m, d2[:, 128 * j:128 * (j + 1)])
    m2 = jnp.minimum(m[:, 0:64], m[:, 64:128])
    thr_ref[0] = jnp.max(m2, axis=1, keepdims=True)


def _d2_thr(centers, xyz, c2, x2):
    return pl.pallas_call(
        _d2_body,
        grid=(B, P // PB),
        in_specs=[
            pl.BlockSpec((1, PB, 3), lambda b, p: (b, p, 0)),
            pl.BlockSpec((1, N, 3), lambda b, p: (b, 0, 0)),
            pl.BlockSpec((1, PB, 1), lambda b, p: (b, p, 0)),
            pl.BlockSpec((1, 1, N), lambda b, p: (b, 0, 0)),
        ],
        out_specs=[
            pl.BlockSpec((1, PB, N), lambda b, p: (b, p, 0)),
            pl.BlockSpec((1, PB, 1), lambda b, p: (b, p, 0)),
        ],
        out_shape=[
            jax.ShapeDtypeStruct((B, P, N), jnp.float32),
            jax.ShapeDtypeStruct((B, P, 1), jnp.float32),
        ],
    )(centers, xyz, c2[:, :, None], x2[:, None, :])


def _sel_body(d2_hbm, thr_hbm, x0_hbm, x1_hbm, x2_hbm, c0_hbm, c1_hbm, c2_hbm,
              out_hbm,
              x0v, x1v, x2v, d2v, ckey, cidx, thrv, c0v, c1v, c2v, outp, oidx):
    wid = lax.axis_index("s") * NC + lax.axis_index("c")
    b = wid // (NW // B)
    r0 = wid * RPW
    inf = jnp.float32(jnp.inf)
    iota16 = lax.iota(jnp.int32, 16)

    pltpu.sync_copy(x0_hbm.at[pl.ds(b * N, N)], x0v)
    pltpu.sync_copy(x1_hbm.at[pl.ds(b * N, N)], x1v)
    pltpu.sync_copy(x2_hbm.at[pl.ds(b * N, N)], x2v)
    pltpu.sync_copy(thr_hbm.at[pl.ds(r0, RPW)], thrv)
    pltpu.sync_copy(c0_hbm.at[pl.ds(r0, RPW)], c0v)
    pltpu.sync_copy(c1_hbm.at[pl.ds(r0, RPW)], c1v)
    pltpu.sync_copy(c2_hbm.at[pl.ds(r0, RPW)], c2v)

    def row_loop(i, _):
        pltpu.sync_copy(d2_hbm.at[pl.ds((r0 + i) * N, N)], d2v)
        bi = jnp.full((16,), i, jnp.int32)
        t = plsc.load_gather(thrv, [bi])
        cc0 = plsc.load_gather(c0v, [bi])
        cc1 = plsc.load_gather(c1v, [bi])
        cc2 = plsc.load_gather(c2v, [bi])

        def compact(j, off_v):
            v = d2v[pl.ds(16 * j, 16)]
            msk = v <= t

            @pl.when(jnp.any(msk))
            def _():
                cum = plsc.cumsum(msk.astype(jnp.int32))
                pos = off_v + cum - 1
                plsc.store_scatter(ckey, [pos], v, mask=msk)
                plsc.store_scatter(cidx, [pos], iota16 + 16 * j, mask=msk)

            return off_v + plsc.all_reduce_population_count(msk)

        off_v = lax.fori_loop(0, N // 16, compact,
                              jnp.zeros((16,), jnp.int32))
        plsc.store_scatter(ckey, [off_v + iota16], jnp.full((16,), inf))
        cnt = jnp.max(off_v)
        nv = (cnt + 15) // 16

        def ext(k, osel):
            def p1(j, mv):
                return jnp.minimum(mv, ckey[pl.ds(16 * j, 16)])

            mv = lax.fori_loop(0, nv, p1, jnp.full((16,), inf))
            mkv = jnp.full((16,), jnp.min(mv))

            def p2(j, iv):
                v = ckey[pl.ds(16 * j, 16)]
                ix = cidx[pl.ds(16 * j, 16)]
                return jnp.minimum(iv, jnp.where(v == mkv, ix, _I32_BIG))

            iv = lax.fori_loop(0, nv, p2, jnp.full((16,), _I32_BIG))
            midx = jnp.min(iv)
            miv = jnp.full((16,), midx)

            def p3(j, _):
                v = ckey[pl.ds(16 * j, 16)]
                ix = cidx[pl.ds(16 * j, 16)]
                hit = (v == mkv) & (ix == miv)

                @pl.when(jnp.any(hit))
                def _():
                    ckey[pl.ds(16 * j, 16)] = jnp.where(hit, inf, v)

                return 0

            lax.fori_loop(0, nv, p3, 0)

            osel = jnp.where(iota16 == k % 16, miv, osel)

            @pl.when(k % 16 == 15)
            def _():
                oidx[pl.ds(16 * (k // 16), 16)] = osel

            return osel

        lax.fori_loop(0, K, ext, jnp.zeros((16,), jnp.int32))

        for g in range(K // 16):
            idxv = oidx[pl.ds(16 * g, 16)]
            gx = plsc.load_gather(x0v, [idxv]) - cc0
            gy = plsc.load_gather(x1v, [idxv]) - cc1
            gz = plsc.load_gather(x2v, [idxv]) - cc2
            base = (iota16 + 16 * g) * 3
            plsc.store_scatter(outp, [base], gx)
            plsc.store_scatter(outp, [base + 1], gy)
            plsc.store_scatter(outp, [base + 2], gz)
        pltpu.sync_copy(outp, out_hbm.at[pl.ds((r0 + i) * 3 * K, 3 * K)])
        return 0

    lax.fori_loop(0, RPW, row_loop, 0)


def _select(d2f, thrf, x0f, x1f, x2f, c0f, c1f, c2f):
    mesh = plsc.VectorSubcoreMesh(core_axis_name="c", subcore_axis_name="s")
    f = functools.partial(
        pl.kernel,
        out_type=jax.ShapeDtypeStruct((B * P * K * 3,), jnp.float32),
        mesh=mesh,
        scratch_types=[
            pltpu.VMEM((N,), jnp.float32),
            pltpu.VMEM((N,), jnp.float32),
            pltpu.VMEM((N,), jnp.float32),
            pltpu.VMEM((N,), jnp.float32),
            pltpu.VMEM((N + 16,), jnp.float32),
            pltpu.VMEM((N + 16,), jnp.int32),
            pltpu.VMEM((RPW,), jnp.float32),
            pltpu.VMEM((RPW,), jnp.float32),
            pltpu.VMEM((RPW,), jnp.float32),
            pltpu.VMEM((RPW,), jnp.float32),
            pltpu.VMEM((3 * K,), jnp.float32),
            pltpu.VMEM((K,), jnp.int32),
        ],
        compiler_params=pltpu.CompilerParams(needs_layout_passes=False),
    )(_sel_body)
    return f(d2f, thrf, x0f, x1f, x2f, c0f, c1f, c2f)


def kernel(xyz):
    x0 = xyz[..., 0]
    x1 = xyz[..., 1]
    x2 = xyz[..., 2]
    c0, c1, c2 = _fps(x0, x1, x2)
    centers = jnp.stack([c0, c1, c2], axis=-1)
    c2n = (c0 * c0 + c1 * c1) + c2 * c2
    x2n = (x0 * x0 + x1 * x1) + x2 * x2
    d2, thr = _d2_thr(centers, xyz, c2n, x2n)
    patches_flat = _select(
        d2.reshape(-1), thr.reshape(-1),
        x0.reshape(-1), x1.reshape(-1), x2.reshape(-1),
        c0.reshape(-1), c1.reshape(-1), c2.reshape(-1))
    patches = patches_flat.reshape(B, P, K, 3)
    return patches[None], centers[None]


# speedup vs baseline: 7.2148x; 1.8687x over previous
import functools

import jax
import jax.numpy as jnp
from jax import lax
from jax.experimental import pallas as pl
from jax.experimental.pallas import tpu as pltpu
from jax.experimental.pallas import tpu_sc as plsc

B = 8
N = 16384
P = 512
K = 64
PB = 128

NC = 2
NS = 16
NW = NC * NS
RPW = (B * P) // NW

_I32_BIG = 2 ** 30


def _fps_body(x0_ref, x1_ref, x2_ref, c0_ref, c1_ref, c2_ref, dist_ref):
    x0 = x0_ref[...]
    x1 = x1_ref[...]
    x2 = x2_ref[...]
    iota_n = lax.broadcasted_iota(jnp.int32, (B, N), 1)
    iota_p = lax.broadcasted_iota(jnp.int32, (B, P), 1)

    p0 = x0[:, 0:1]
    p1 = x1[:, 0:1]
    p2 = x2[:, 0:1]
    c0_ref[...] = jnp.where(iota_p == 0, p0, 0.0)
    c1_ref[...] = jnp.where(iota_p == 0, p1, 0.0)
    c2_ref[...] = jnp.where(iota_p == 0, p2, 0.0)
    d0 = x0 - p0
    d1 = x1 - p1
    d2 = x2 - p2
    dist_ref[...] = (d0 * d0 + d1 * d1) + d2 * d2

    def step(i, _):
        dist = dist_ref[...]
        m = jnp.max(dist, axis=1, keepdims=True)
        idx = jnp.min(jnp.where(dist == m, iota_n, N), axis=1, keepdims=True)
        sel = iota_n == idx
        p0 = jnp.sum(jnp.where(sel, x0, 0.0), axis=1, keepdims=True)
        p1 = jnp.sum(jnp.where(sel, x1, 0.0), axis=1, keepdims=True)
        p2 = jnp.sum(jnp.where(sel, x2, 0.0), axis=1, keepdims=True)
        hit = iota_p == i
        c0_ref[...] = jnp.where(hit, p0, c0_ref[...])
        c1_ref[...] = jnp.where(hit, p1, c1_ref[...])
        c2_ref[...] = jnp.where(hit, p2, c2_ref[...])
        e0 = x0 - p0
        e1 = x1 - p1
        e2 = x2 - p2
        nd = (e0 * e0 + e1 * e1) + e2 * e2
        dist_ref[...] = jnp.minimum(dist, nd)
        return 0

    lax.fori_loop(1, P, step, 0)


def _fps(x0, x1, x2):
    return pl.pallas_call(
        _fps_body,
        out_shape=[jax.ShapeDtypeStruct((B, P), jnp.float32)] * 3,
        scratch_shapes=[pltpu.VMEM((B, N), jnp.float32)],
    )(x0, x1, x2)


def _d2_body(c_ref, x_ref, c2_ref, x2_ref, d2_ref, thr_ref):
    cross = lax.dot_general(c_ref[0], x_ref[0], (((1,), (1,)), ((), ())))
    d2 = (c2_ref[0] + x2_ref[0]) - 2.0 * cross
    d2_ref[0] = d2
    m = d2[:, 0:128]
    for j in range(1, N // 128):
        m = jnp.minimum(m, d2[:, 128 * j:128 * (j + 1)])
    m2 = jnp.minimum(m[:, 0:64], m[:, 64:128])
    thr_ref[0] = jnp.max(m2, axis=1, keepdims=True)


def _d2_thr(centers, xyz, c2, x2):
    return pl.pallas_call(
        _d2_body,
        grid=(B, P // PB),
        in_specs=[
            pl.BlockSpec((1, PB, 3), lambda b, p: (b, p, 0)),
            pl.BlockSpec((1, N, 3), lambda b, p: (b, 0, 0)),
            pl.BlockSpec((1, PB, 1), lambda b, p: (b, p, 0)),
            pl.BlockSpec((1, 1, N), lambda b, p: (b, 0, 0)),
        ],
        out_specs=[
            pl.BlockSpec((1, PB, N), lambda b, p: (b, p, 0)),
            pl.BlockSpec((1, PB, 1), lambda b, p: (b, p, 0)),
        ],
        out_shape=[
            jax.ShapeDtypeStruct((B, P, N), jnp.float32),
            jax.ShapeDtypeStruct((B, P, 1), jnp.float32),
        ],
    )(centers, xyz, c2[:, :, None], x2[:, None, :])


OB = 8


def _sel_body(d2_hbm, thr_hbm, x0_hbm, x1_hbm, x2_hbm, c0_hbm, c1_hbm, c2_hbm,
              out_hbm,
              x0v, x1v, x2v, d2v, ckey, cidx, thrv, c0v, c1v, c2v, outp, oidx,
              sem):
    wid = lax.axis_index("s") * NC + lax.axis_index("c")
    b = wid // (NW // B)
    r0 = wid * RPW
    inf = jnp.float32(jnp.inf)
    iota16 = lax.iota(jnp.int32, 16)
    infv = jnp.full((16,), inf)

    pltpu.sync_copy(x0_hbm.at[pl.ds(b * N, N)], x0v)
    pltpu.sync_copy(x1_hbm.at[pl.ds(b * N, N)], x1v)
    pltpu.sync_copy(x2_hbm.at[pl.ds(b * N, N)], x2v)
    pltpu.sync_copy(thr_hbm.at[pl.ds(r0, RPW)], thrv)
    pltpu.sync_copy(c0_hbm.at[pl.ds(r0, RPW)], c0v)
    pltpu.sync_copy(c1_hbm.at[pl.ds(r0, RPW)], c1v)
    pltpu.sync_copy(c2_hbm.at[pl.ds(r0, RPW)], c2v)
    pltpu.async_copy(d2_hbm.at[pl.ds(r0 * N, N)], d2v, sem)

    def row_loop(i, _):
        pltpu.make_async_copy(d2_hbm.at[pl.ds(0, N)], d2v, sem).wait()
        bi = jnp.full((16,), i, jnp.int32)
        t = plsc.load_gather(thrv, [bi])
        cc0 = plsc.load_gather(c0v, [bi])
        cc1 = plsc.load_gather(c1v, [bi])
        cc2 = plsc.load_gather(c2v, [bi])

        def compact(j, off_v):
            for u in range(4):
                v = d2v[pl.ds(64 * j + 16 * u, 16)]
                msk = v <= t

                @pl.when(jnp.any(msk))
                def _():
                    cum = plsc.cumsum(jnp.where(msk, 1, 0))
                    pos = off_v + cum - 1
                    plsc.store_scatter(ckey, [pos], v, mask=msk)
                    plsc.store_scatter(cidx, [pos],
                                       iota16 + (64 * j + 16 * u), mask=msk)

                off_v = off_v + plsc.all_reduce_population_count(msk)
            return off_v

        off_v = lax.fori_loop(0, N // 64, compact,
                              jnp.zeros((16,), jnp.int32))
        for u in range(4):
            plsc.store_scatter(ckey, [off_v + iota16 + 16 * u], infv)
        cnt = jnp.max(off_v)
        nvu = (cnt + 63) // 64

        @pl.when(i + 1 < RPW)
        def _():
            pltpu.async_copy(d2_hbm.at[pl.ds((r0 + i + 1) * N, N)], d2v, sem)

        def ext(k, osel):
            def p1(j, mv):
                m0 = jnp.minimum(ckey[pl.ds(64 * j, 16)],
                                 ckey[pl.ds(64 * j + 16, 16)])
                m1 = jnp.minimum(ckey[pl.ds(64 * j + 32, 16)],
                                 ckey[pl.ds(64 * j + 48, 16)])
                return jnp.minimum(mv, jnp.minimum(m0, m1))

            mv = lax.fori_loop(0, nvu, p1, infv)
            mkv = jnp.full((16,), jnp.min(mv))

            def p2(j, pv):
                base = 64 * j + iota16
                for u in range(4):
                    v = ckey[pl.ds(64 * j + 16 * u, 16)]
                    pv = jnp.minimum(
                        pv, jnp.where(v == mkv, base + 16 * u, _I32_BIG))
                return pv

            pv = lax.fori_loop(0, nvu, p2, jnp.full((16,), _I32_BIG))
            mpos = jnp.full((16,), jnp.min(pv))
            widx = plsc.load_gather(cidx, [mpos])
            plsc.store_scatter(ckey, [mpos], infv)

            osel = jnp.where(iota16 == k % 16, widx, osel)

            @pl.when(k % 16 == 15)
            def _():
                oidx[pl.ds(16 * (k // 16), 16)] = osel

            return osel

        lax.fori_loop(0, K, ext, jnp.zeros((16,), jnp.int32))

        ob = (i % OB) * 3 * K
        for g in range(K // 16):
            idxv = oidx[pl.ds(16 * g, 16)]
            gx = plsc.load_gather(x0v, [idxv]) - cc0
            gy = plsc.load_gather(x1v, [idxv]) - cc1
            gz = plsc.load_gather(x2v, [idxv]) - cc2
            base = (iota16 + 16 * g) * 3 + ob
            plsc.store_scatter(outp, [base], gx)
            plsc.store_scatter(outp, [base + 1], gy)
            plsc.store_scatter(outp, [base + 2], gz)

        @pl.when(i % OB == OB - 1)
        def _():
            pltpu.sync_copy(
                outp,
                out_hbm.at[pl.ds((r0 + i - (OB - 1)) * 3 * K, OB * 3 * K)])

        return 0

    lax.fori_loop(0, RPW, row_loop, 0)


def _select(d2f, thrf, x0f, x1f, x2f, c0f, c1f, c2f):
    mesh = plsc.VectorSubcoreMesh(core_axis_name="c", subcore_axis_name="s")
    f = functools.partial(
        pl.kernel,
        out_type=jax.ShapeDtypeStruct((B * P * K * 3,), jnp.float32),
        mesh=mesh,
        scratch_types=[
            pltpu.VMEM((N,), jnp.float32),
            pltpu.VMEM((N,), jnp.float32),
            pltpu.VMEM((N,), jnp.float32),
            pltpu.VMEM((N,), jnp.float32),
            pltpu.VMEM((N + 64,), jnp.float32),
            pltpu.VMEM((N + 64,), jnp.int32),
            pltpu.VMEM((RPW,), jnp.float32),
            pltpu.VMEM((RPW,), jnp.float32),
            pltpu.VMEM((RPW,), jnp.float32),
            pltpu.VMEM((RPW,), jnp.float32),
            pltpu.VMEM((OB * 3 * K,), jnp.float32),
            pltpu.VMEM((K,), jnp.int32),
            pltpu.SemaphoreType.DMA,
        ],
        compiler_params=pltpu.CompilerParams(needs_layout_passes=False),
    )(_sel_body)
    return f(d2f, thrf, x0f, x1f, x2f, c0f, c1f, c2f)


def kernel(xyz):
    x0 = xyz[..., 0]
    x1 = xyz[..., 1]
    x2 = xyz[..., 2]
    c0, c1, c2 = _fps(x0, x1, x2)
    centers = jnp.stack([c0, c1, c2], axis=-1)
    c2n = (c0 * c0 + c1 * c1) + c2 * c2
    x2n = (x0 * x0 + x1 * x1) + x2 * x2
    d2, thr = _d2_thr(centers, xyz, c2n, x2n)
    patches_flat = _select(
        d2.reshape(-1), thr.reshape(-1),
        x0.reshape(-1), x1.reshape(-1), x2.reshape(-1),
        c0.reshape(-1), c1.reshape(-1), c2.reshape(-1))
    patches = patches_flat.reshape(B, P, K, 3)
    return patches[None], centers[None]


# speedup vs baseline: 11.6064x; 1.6087x over previous
import functools

import jax
import jax.numpy as jnp
from jax import lax
from jax.experimental import pallas as pl
from jax.experimental.pallas import tpu as pltpu
from jax.experimental.pallas import tpu_sc as plsc

B = 8
N = 16384
P = 512
K = 64
PB = 128

NC = 2
NS = 16
NW = NC * NS
RPW = (B * P) // NW

_I32_BIG = 2 ** 30


def _fps_body(x0_ref, x1_ref, x2_ref, c0_ref, c1_ref, c2_ref, dist_ref):
    x0 = x0_ref[...]
    x1 = x1_ref[...]
    x2 = x2_ref[...]
    iota_n = lax.broadcasted_iota(jnp.int32, (B, N), 1)
    iota_p = lax.broadcasted_iota(jnp.int32, (B, P), 1)

    p0 = x0[:, 0:1]
    p1 = x1[:, 0:1]
    p2 = x2[:, 0:1]
    c0_ref[...] = jnp.where(iota_p == 0, p0, 0.0)
    c1_ref[...] = jnp.where(iota_p == 0, p1, 0.0)
    c2_ref[...] = jnp.where(iota_p == 0, p2, 0.0)
    d0 = x0 - p0
    d1 = x1 - p1
    d2 = x2 - p2
    dist_ref[...] = (d0 * d0 + d1 * d1) + d2 * d2

    def step(i, _):
        dist = dist_ref[...]
        m = jnp.max(dist, axis=1, keepdims=True)
        idx = jnp.min(jnp.where(dist == m, iota_n, N), axis=1, keepdims=True)
        sel = iota_n == idx
        p0 = jnp.sum(jnp.where(sel, x0, 0.0), axis=1, keepdims=True)
        p1 = jnp.sum(jnp.where(sel, x1, 0.0), axis=1, keepdims=True)
        p2 = jnp.sum(jnp.where(sel, x2, 0.0), axis=1, keepdims=True)
        hit = iota_p == i
        c0_ref[...] = jnp.where(hit, p0, c0_ref[...])
        c1_ref[...] = jnp.where(hit, p1, c1_ref[...])
        c2_ref[...] = jnp.where(hit, p2, c2_ref[...])
        e0 = x0 - p0
        e1 = x1 - p1
        e2 = x2 - p2
        nd = (e0 * e0 + e1 * e1) + e2 * e2
        dist_ref[...] = jnp.minimum(dist, nd)
        return 0

    lax.fori_loop(1, P, step, 0)


def _fps(x0, x1, x2):
    return pl.pallas_call(
        _fps_body,
        out_shape=[jax.ShapeDtypeStruct((B, P), jnp.float32)] * 3,
        scratch_shapes=[pltpu.VMEM((B, N), jnp.float32)],
    )(x0, x1, x2)


def _d2_body(c_ref, x_ref, c2_ref, x2_ref, d2_ref, thr_ref):
    cross = lax.dot_general(c_ref[0], x_ref[0], (((1,), (1,)), ((), ())))
    d2 = (c2_ref[0] + x2_ref[0]) - 2.0 * cross
    d2_ref[0] = d2
    m = d2[:, 0:128]
    for j in range(1, N // 128):
        m = jnp.minimum(m, d2[:, 128 * j:128 * (j + 1)])
    m2 = jnp.minimum(m[:, 0:64], m[:, 64:128])
    thr_ref[0] = jnp.max(m2, axis=1, keepdims=True)


def _d2_thr(centers, xyz, c2, x2):
    return pl.pallas_call(
        _d2_body,
        grid=(B, P // PB),
        in_specs=[
            pl.BlockSpec((1, PB, 3), lambda b, p: (b, p, 0)),
            pl.BlockSpec((1, N, 3), lambda b, p: (b, 0, 0)),
            pl.BlockSpec((1, PB, 1), lambda b, p: (b, p, 0)),
            pl.BlockSpec((1, 1, N), lambda b, p: (b, 0, 0)),
        ],
        out_specs=[
            pl.BlockSpec((1, PB, N), lambda b, p: (b, p, 0)),
            pl.BlockSpec((1, PB, 1), lambda b, p: (b, p, 0)),
        ],
        out_shape=[
            jax.ShapeDtypeStruct((B, P, N), jnp.float32),
            jax.ShapeDtypeStruct((B, P, 1), jnp.float32),
        ],
    )(centers, xyz, c2[:, :, None], x2[:, None, :])


OB = 8


def _sel_body(d2_hbm, thr_hbm, x0_hbm, x1_hbm, x2_hbm, c0_hbm, c1_hbm, c2_hbm,
              out_hbm,
              x0v, x1v, x2v, d2v, ckey, cidx, thrv, c0v, c1v, c2v, outp, oidx,
              sem):
    wid = lax.axis_index("s") * NC + lax.axis_index("c")
    b = wid // (NW // B)
    r0 = wid * RPW
    inf = jnp.float32(jnp.inf)
    iota16 = lax.iota(jnp.int32, 16)
    infv = jnp.full((16,), inf)

    pltpu.sync_copy(x0_hbm.at[pl.ds(b * N, N)], x0v)
    pltpu.sync_copy(x1_hbm.at[pl.ds(b * N, N)], x1v)
    pltpu.sync_copy(x2_hbm.at[pl.ds(b * N, N)], x2v)
    pltpu.sync_copy(thr_hbm.at[pl.ds(r0, RPW)], thrv)
    pltpu.sync_copy(c0_hbm.at[pl.ds(r0, RPW)], c0v)
    pltpu.sync_copy(c1_hbm.at[pl.ds(r0, RPW)], c1v)
    pltpu.sync_copy(c2_hbm.at[pl.ds(r0, RPW)], c2v)
    pltpu.async_copy(d2_hbm.at[pl.ds(r0 * N, N)], d2v, sem)

    def row_loop(i, _):
        pltpu.make_async_copy(d2_hbm.at[pl.ds(0, N)], d2v, sem).wait()
        bi = jnp.full((16,), i, jnp.int32)
        t = plsc.load_gather(thrv, [bi])
        cc0 = plsc.load_gather(c0v, [bi])
        cc1 = plsc.load_gather(c1v, [bi])
        cc2 = plsc.load_gather(c2v, [bi])

        def compact(j, off_v):
            for u in range(4):
                v = d2v[pl.ds(64 * j + 16 * u, 16)]
                msk = v <= t
                cum = plsc.cumsum(jnp.where(msk, 1, 0))
                pos = off_v + cum - 1
                plsc.store_scatter(ckey, [pos], v, mask=msk)
                plsc.store_scatter(cidx, [pos],
                                   iota16 + (64 * j + 16 * u), mask=msk)
                off_v = off_v + plsc.all_reduce_population_count(msk)
            return off_v

        off_v = lax.fori_loop(0, N // 64, compact,
                              jnp.zeros((16,), jnp.int32))
        for u in range(4):
            plsc.store_scatter(ckey, [off_v + iota16 + 16 * u], infv)
        cnt = jnp.max(off_v)
        nvu = (cnt + 63) // 64

        @pl.when(i + 1 < RPW)
        def _():
            pltpu.async_copy(d2_hbm.at[pl.ds((r0 + i + 1) * N, N)], d2v, sem)

        def vsplat_min(v, big):
            for sh in (8, 4, 2, 1):
                perm = (iota16 + sh) & 15
                v = jnp.minimum(
                    v, v.at[perm].get(mode="promise_in_bounds"))
            return v

        def ext(k, osel):
            def p1(j, mv):
                m0 = jnp.minimum(ckey[pl.ds(64 * j, 16)],
                                 ckey[pl.ds(64 * j + 16, 16)])
                m1 = jnp.minimum(ckey[pl.ds(64 * j + 32, 16)],
                                 ckey[pl.ds(64 * j + 48, 16)])
                return jnp.minimum(mv, jnp.minimum(m0, m1))

            mv = lax.fori_loop(0, nvu, p1, infv)
            mkv = vsplat_min(mv, inf)

            def p2(j, pv):
                base = 64 * j + iota16
                for u in range(4):
                    v = ckey[pl.ds(64 * j + 16 * u, 16)]
                    pv = jnp.minimum(
                        pv, jnp.where(v == mkv, base + 16 * u, _I32_BIG))
                return pv

            pv = lax.fori_loop(0, nvu, p2, jnp.full((16,), _I32_BIG))
            mpos = vsplat_min(pv, _I32_BIG)
            widx = plsc.load_gather(cidx, [mpos])
            plsc.store_scatter(ckey, [mpos], infv)

            osel = jnp.where(iota16 == k % 16, widx, osel)

            @pl.when(k % 16 == 15)
            def _():
                oidx[pl.ds(16 * (k // 16), 16)] = osel

            return osel

        lax.fori_loop(0, K, ext, jnp.zeros((16,), jnp.int32))

        ob = (i % OB) * 3 * K
        for g in range(K // 16):
            idxv = oidx[pl.ds(16 * g, 16)]
            gx = plsc.load_gather(x0v, [idxv]) - cc0
            gy = plsc.load_gather(x1v, [idxv]) - cc1
            gz = plsc.load_gather(x2v, [idxv]) - cc2
            base = (iota16 + 16 * g) * 3 + ob
            plsc.store_scatter(outp, [base], gx)
            plsc.store_scatter(outp, [base + 1], gy)
            plsc.store_scatter(outp, [base + 2], gz)

        @pl.when(i % OB == OB - 1)
        def _():
            pltpu.sync_copy(
                outp,
                out_hbm.at[pl.ds((r0 + i - (OB - 1)) * 3 * K, OB * 3 * K)])

        return 0

    lax.fori_loop(0, RPW, row_loop, 0)


def _select(d2f, thrf, x0f, x1f, x2f, c0f, c1f, c2f):
    mesh = plsc.VectorSubcoreMesh(core_axis_name="c", subcore_axis_name="s")
    f = functools.partial(
        pl.kernel,
        out_type=jax.ShapeDtypeStruct((B * P * K * 3,), jnp.float32),
        mesh=mesh,
        scratch_types=[
            pltpu.VMEM((N,), jnp.float32),
            pltpu.VMEM((N,), jnp.float32),
            pltpu.VMEM((N,), jnp.float32),
            pltpu.VMEM((N,), jnp.float32),
            pltpu.VMEM((N + 64,), jnp.float32),
            pltpu.VMEM((N + 64,), jnp.int32),
            pltpu.VMEM((RPW,), jnp.float32),
            pltpu.VMEM((RPW,), jnp.float32),
            pltpu.VMEM((RPW,), jnp.float32),
            pltpu.VMEM((RPW,), jnp.float32),
            pltpu.VMEM((OB * 3 * K,), jnp.float32),
            pltpu.VMEM((K,), jnp.int32),
            pltpu.SemaphoreType.DMA,
        ],
        compiler_params=pltpu.CompilerParams(needs_layout_passes=False),
    )(_sel_body)
    return f(d2f, thrf, x0f, x1f, x2f, c0f, c1f, c2f)


def kernel(xyz):
    x0 = xyz[..., 0]
    x1 = xyz[..., 1]
    x2 = xyz[..., 2]
    c0, c1, c2 = _fps(x0, x1, x2)
    centers = jnp.stack([c0, c1, c2], axis=-1)
    c2n = (c0 * c0 + c1 * c1) + c2 * c2
    x2n = (x0 * x0 + x1 * x1) + x2 * x2
    d2, thr = _d2_thr(centers, xyz, c2n, x2n)
    patches_flat = _select(
        d2.reshape(-1), thr.reshape(-1),
        x0.reshape(-1), x1.reshape(-1), x2.reshape(-1),
        c0.reshape(-1), c1.reshape(-1), c2.reshape(-1))
    patches = patches_flat.reshape(B, P, K, 3)
    return patches[None], centers[None]
